# baseline (speedup 1.0000x reference)
import functools

import jax
import jax.numpy as jnp
from jax import lax
from jax.experimental import pallas as pl
from jax.experimental.pallas import tpu as pltpu
from jax.experimental.pallas import tpu_sc as plsc

_DEFAULT_INCREASE = 2048
_N, _D = 131072, 256
_NC, _NS = 2, 16
_NW = _NC * _NS
_ROWS_PER_W = _N // _NW
_CHUNK = 128
_NBUF = 3
_NCHUNKS = _ROWS_PER_W // _CHUNK

_mesh = plsc.VectorSubcoreMesh(core_axis_name="c", subcore_axis_name="s")


@functools.partial(
    pl.kernel,
    mesh=_mesh,
    out_type=jax.ShapeDtypeStruct((_N, _D), jnp.float32),
    scratch_types=(
        [pltpu.VMEM((_NBUF, _CHUNK, _D), jnp.float32)]
        + [pltpu.SemaphoreType.DMA] * (2 * _NBUF)
    ),
)
def _sc_copy(x_hbm, out_hbm, buf, *sems):
    in_sems, out_sems = sems[:_NBUF], sems[_NBUF:]
    wid = lax.axis_index("s") * _NC + lax.axis_index("c")
    base = wid * _ROWS_PER_W

    def start_in(chunk):
        slot = chunk % _NBUF
        c = pltpu.make_async_copy(
            x_hbm.at[pl.ds(base + chunk * _CHUNK, _CHUNK)],
            buf.at[slot],
            in_sems[slot],
        )
        c.start()
        return c

    def start_out(chunk):
        slot = chunk % _NBUF
        c = pltpu.make_async_copy(
            buf.at[slot],
            out_hbm.at[pl.ds(base + chunk * _CHUNK, _CHUNK)],
            out_sems[slot],
        )
        c.start()
        return c

    ins = [None] * _NBUF
    for c in range(min(2, _NCHUNKS)):
        ins[c % _NBUF] = start_in(c)
    outs = [None] * _NCHUNKS
    for chunk in range(_NCHUNKS):
        nxt = chunk + 2
        if nxt < _NCHUNKS:
            prev = nxt - _NBUF
            if prev >= 0:
                outs[prev].wait()
            ins[nxt % _NBUF] = start_in(nxt)
        ins[chunk % _NBUF].wait()
        outs[chunk] = start_out(chunk)
    for chunk in range(max(0, _NCHUNKS - _NBUF), _NCHUNKS):
        outs[chunk].wait()


def kernel(x):
    n, d = x.shape
    padded = -(-n // _DEFAULT_INCREASE) * _DEFAULT_INCREASE
    assert (padded, d) == (_N, _D), "fixed problem shape"
    return _sc_copy(x)


# speedup vs baseline: 1.0154x; 1.0154x over previous
import functools

import jax
import jax.numpy as jnp
from jax import lax
from jax.experimental import pallas as pl
from jax.experimental.pallas import tpu as pltpu
from jax.experimental.pallas import tpu_sc as plsc

_DEFAULT_INCREASE = 2048
_N, _D = 131072, 256
_NC, _NS = 2, 16
_NW = _NC * _NS
_ROWS_PER_W = _N // _NW
_CHUNK = 248
_NBUF = 2
_CHUNK_OFFS = list(range(0, _ROWS_PER_W - _CHUNK + 1, _CHUNK))
_CHUNK_SIZES = [_CHUNK] * len(_CHUNK_OFFS)
_TAIL = _ROWS_PER_W - len(_CHUNK_OFFS) * _CHUNK
if _TAIL:
    _CHUNK_OFFS.append(len(_CHUNK_OFFS) * _CHUNK)
    _CHUNK_SIZES.append(_TAIL)
_NCHUNKS = len(_CHUNK_OFFS)

_mesh = plsc.VectorSubcoreMesh(core_axis_name="c", subcore_axis_name="s")


@functools.partial(
    pl.kernel,
    mesh=_mesh,
    out_type=jax.ShapeDtypeStruct((_N, _D), jnp.float32),
    scratch_types=(
        [pltpu.VMEM((_NBUF, _CHUNK, _D), jnp.float32)]
        + [pltpu.SemaphoreType.DMA] * (2 * _NBUF)
    ),
)
def _sc_copy(x_hbm, out_hbm, buf, *sems):
    in_sems, out_sems = sems[:_NBUF], sems[_NBUF:]
    wid = lax.axis_index("s") * _NC + lax.axis_index("c")
    base = wid * _ROWS_PER_W

    def start_in(chunk):
        slot = chunk % _NBUF
        sz = _CHUNK_SIZES[chunk]
        c = pltpu.make_async_copy(
            x_hbm.at[pl.ds(base + _CHUNK_OFFS[chunk], sz)],
            buf.at[slot, pl.ds(0, sz)],
            in_sems[slot],
        )
        c.start()
        return c

    def start_out(chunk):
        slot = chunk % _NBUF
        sz = _CHUNK_SIZES[chunk]
        c = pltpu.make_async_copy(
            buf.at[slot, pl.ds(0, sz)],
            out_hbm.at[pl.ds(base + _CHUNK_OFFS[chunk], sz)],
            out_sems[slot],
        )
        c.start()
        return c

    ins = [None] * _NBUF
    for c in range(min(_NBUF, _NCHUNKS)):
        ins[c % _NBUF] = start_in(c)
    outs = [None] * _NCHUNKS
    for chunk in range(_NCHUNKS):
        ins[chunk % _NBUF].wait()
        outs[chunk] = start_out(chunk)
        nxt = chunk + _NBUF
        if nxt < _NCHUNKS:
            outs[chunk].wait()
            ins[nxt % _NBUF] = start_in(nxt)
    for chunk in range(max(0, _NCHUNKS - _NBUF), _NCHUNKS):
        outs[chunk].wait()


def kernel(x):
    n, d = x.shape
    padded = -(-n // _DEFAULT_INCREASE) * _DEFAULT_INCREASE
    assert (padded, d) == (_N, _D), "fixed problem shape"
    return _sc_copy(x)
